# Initial kernel scaffold; baseline (speedup 1.0000x reference)
import jax
import jax.numpy as jnp
from jax.experimental import pallas as pl


def kernel(x, expert_ids, fused_wi, fused_wo):
    raise NotImplementedError("write your pallas kernel here")


# speedup vs baseline: 2.5035x; 2.5035x over previous
import functools

import jax
import jax.numpy as jnp
from jax import lax
from jax.experimental import pallas as pl
from jax.experimental.pallas import tpu as pltpu
from jax.experimental.pallas import tpu_sc as plsc

E = 8
D = 768
F = 3072
T = 2048

TILE = 256
FT = 512
NF = F // FT
WMAX = T // TILE + E - 1
PAD = WMAX * TILE

NC = 2
NS = 16
NW = NC * NS
TPW = T // NW
LANE = 16

_mesh = plsc.VectorSubcoreMesh(
    core_axis_name="c", subcore_axis_name="s", num_cores=NC, num_subcores=NS)


def _wid():
    return lax.axis_index("s") * NC + lax.axis_index("c")


@functools.partial(
    pl.kernel,
    out_type=jax.ShapeDtypeStruct((NW, LANE), jnp.int32),
    mesh=_mesh,
    scratch_types=[
        pltpu.VMEM((TPW,), jnp.int32),
        pltpu.VMEM((LANE,), jnp.int32),
    ],
    compiler_params=pltpu.CompilerParams(needs_layout_passes=False),
)
def _sc_hist(ids_hbm, hist_hbm, ids_v, cnt_v):
    w = _wid()
    pltpu.sync_copy(ids_hbm.at[pl.ds(w * TPW, TPW)], ids_v)
    cnt_v[...] = jnp.zeros((LANE,), jnp.int32)
    for v in range(TPW // LANE):
        idv = ids_v[pl.ds(v * LANE, LANE)]
        occ, last = plsc.scan_count(idv)
        base = plsc.load_gather(cnt_v, [idv])
        plsc.store_scatter(cnt_v, [idv], base + occ, mask=last)
    pltpu.sync_copy(cnt_v, hist_hbm.at[w])


@functools.partial(
    pl.kernel,
    out_type=(
        jax.ShapeDtypeStruct((T,), jnp.int32),
        jax.ShapeDtypeStruct((PAD, D), jnp.float32),
    ),
    mesh=_mesh,
    scratch_types=[
        pltpu.VMEM((TPW,), jnp.int32),
        pltpu.VMEM((LANE,), jnp.int32),
        pltpu.VMEM((TPW,), jnp.int32),
        pltpu.VMEM((TPW, D), jnp.float32),
        pltpu.SemaphoreType.DMA,
    ],
    compiler_params=pltpu.CompilerParams(needs_layout_passes=False),
)
def _sc_rank_scatter(x_hbm, ids_hbm, start_hbm, rank_hbm, xs_hbm,
                     ids_v, start_v, rank_v, xv, sem):
    w = _wid()
    pltpu.sync_copy(ids_hbm.at[pl.ds(w * TPW, TPW)], ids_v)
    pltpu.sync_copy(start_hbm.at[w], start_v)
    for v in range(TPW // LANE):
        idv = ids_v[pl.ds(v * LANE, LANE)]
        occ, last = plsc.scan_count(idv)
        base = plsc.load_gather(start_v, [idv])
        rank = base + occ - 1
        rank_v[pl.ds(v * LANE, LANE)] = rank
        plsc.store_scatter(start_v, [idv], rank + 1, mask=last)
    pltpu.sync_copy(rank_v, rank_hbm.at[pl.ds(w * TPW, TPW)])
    pltpu.sync_copy(x_hbm.at[pl.ds(w * TPW, TPW)], xv)
    pltpu.async_copy(xv, xs_hbm.at[rank_v], sem).wait()


@functools.partial(
    pl.kernel,
    out_type=jax.ShapeDtypeStruct((T, D), jnp.float32),
    mesh=_mesh,
    scratch_types=[
        pltpu.VMEM((TPW,), jnp.int32),
        pltpu.VMEM((TPW, D), jnp.float32),
        pltpu.SemaphoreType.DMA,
    ],
    compiler_params=pltpu.CompilerParams(needs_layout_passes=False),
)
def _sc_gather_out(ys_hbm, rank_hbm, out_hbm, idx_v, rows_v, sem):
    w = _wid()
    pltpu.sync_copy(rank_hbm.at[pl.ds(w * TPW, TPW)], idx_v)
    pltpu.async_copy(ys_hbm.at[idx_v], rows_v, sem).wait()
    pltpu.sync_copy(rows_v, out_hbm.at[pl.ds(w * TPW, TPW)])


def _gemm_body(gids_ref, nreal_ref, x_ref, wi_ref, wo_ref, out_ref):
    f = pl.program_id(0)
    w = pl.program_id(1)

    @pl.when(w < nreal_ref[0])
    def _():
        h = jnp.dot(x_ref[...], wi_ref[0], preferred_element_type=jnp.float32)
        h = jnp.maximum(h, 0.0)
        acc = jnp.dot(h, wo_ref[0], preferred_element_type=jnp.float32)
        base = w * TILE

        @pl.when(f == 0)
        def _():
            out_ref[pl.ds(base, TILE), :] = acc

        @pl.when(f > 0)
        def _():
            out_ref[pl.ds(base, TILE), :] += acc


def _grouped_gemm(gids, nreal, xs, fused_wi, fused_wo):
    grid_spec = pltpu.PrefetchScalarGridSpec(
        num_scalar_prefetch=2,
        grid=(NF, WMAX),
        in_specs=[
            pl.BlockSpec((TILE, D), lambda f, w, g, n: (w, 0)),
            pl.BlockSpec((1, D, FT), lambda f, w, g, n: (g[w], 0, f)),
            pl.BlockSpec((1, FT, D), lambda f, w, g, n: (g[w], f, 0)),
        ],
        out_specs=pl.BlockSpec((PAD, D), lambda f, w, g, n: (0, 0)),
    )
    return pl.pallas_call(
        _gemm_body,
        grid_spec=grid_spec,
        out_shape=jax.ShapeDtypeStruct((PAD, D), jnp.float32),
    )(gids, nreal, xs, fused_wi, fused_wo)


def kernel(x, expert_ids, fused_wi, fused_wo):
    ids = expert_ids.astype(jnp.int32)

    hist = _sc_hist(ids)
    hist_e = hist[:, :E]
    counts = jnp.sum(hist_e, axis=0)
    tiles = (counts + TILE - 1) // TILE
    cum_tiles = jnp.cumsum(tiles)
    padded_base = (cum_tiles - tiles) * TILE
    excl = jnp.cumsum(hist_e, axis=0) - hist_e
    start = padded_base[None, :] + excl
    start = jnp.pad(start, ((0, 0), (0, LANE - E)))

    rank, xs = _sc_rank_scatter(x, ids, start)

    gids = jnp.searchsorted(cum_tiles, jnp.arange(WMAX, dtype=jnp.int32),
                            side="right").astype(jnp.int32)
    gids = jnp.minimum(gids, E - 1)
    nreal = cum_tiles[-1:].astype(jnp.int32)

    ys = _grouped_gemm(gids, nreal, xs, fused_wi, fused_wo)
    return _sc_gather_out(ys, rank)
